# baseline (speedup 1.0000x reference)
import functools

import jax
import jax.numpy as jnp
import numpy as np
from jax import lax
from jax.experimental import pallas as pl
from jax.experimental.pallas import tpu as pltpu
from jax.experimental.pallas import tpu_sc as plsc

NC, NS = 2, 16
NW = NC * NS
GATHER_CHUNK = 32
TOPK_DENSITY = 0.11


def _sc_gather(ids, table):
    n = ids.shape[0]
    _, d = table.shape
    b_per_w = n // NW
    n_chunks = b_per_w // GATHER_CHUNK
    mesh = plsc.VectorSubcoreMesh(core_axis_name="c", subcore_axis_name="s")

    @functools.partial(
        pl.kernel,
        mesh=mesh,
        out_type=jax.ShapeDtypeStruct((n, d), jnp.float32),
        scratch_types=[
            pltpu.VMEM((GATHER_CHUNK,), jnp.int32),
            pltpu.VMEM((GATHER_CHUNK,), jnp.int32),
            pltpu.VMEM((GATHER_CHUNK, d), jnp.float32),
            pltpu.VMEM((GATHER_CHUNK, d), jnp.float32),
            pltpu.SemaphoreType.DMA,
            pltpu.SemaphoreType.DMA,
            pltpu.SemaphoreType.DMA,
            pltpu.SemaphoreType.DMA,
        ],
    )
    def gather_kernel(ids_hbm, table_hbm, out_hbm,
                      idx0, idx1, rows0, rows1, gs0, gs1, ws0, ws1):
        wid = lax.axis_index("s") * NC + lax.axis_index("c")
        base = wid * b_per_w
        idx = (idx0, idx1)
        rows = (rows0, rows1)
        gsem = (gs0, gs1)
        wsem = (ws0, ws1)
        pltpu.sync_copy(ids_hbm.at[pl.ds(base, GATHER_CHUNK)], idx0)
        gathers = {0: pltpu.async_copy(table_hbm.at[idx0], rows0, gs0)}
        writes = {}
        for i in range(n_chunks):
            cur = i % 2
            nxt = (i + 1) % 2
            if i + 1 < n_chunks:
                if i - 1 >= 0:
                    writes[i - 1].wait()
                off_n = base + (i + 1) * GATHER_CHUNK
                pltpu.sync_copy(ids_hbm.at[pl.ds(off_n, GATHER_CHUNK)],
                                idx[nxt])
                gathers[i + 1] = pltpu.async_copy(
                    table_hbm.at[idx[nxt]], rows[nxt], gsem[nxt])
            gathers[i].wait()
            off = base + i * GATHER_CHUNK
            writes[i] = pltpu.async_copy(
                rows[cur], out_hbm.at[pl.ds(off, GATHER_CHUNK)], wsem[cur])
        writes[n_chunks - 2].wait()
        writes[n_chunks - 1].wait()

    return gather_kernel(ids, table)


def _ln_topk_body(x_ref, g_ref, b_ref, spikes_ref, xn_ref, *, k):
    x = x_ref[...]
    d = x.shape[1]
    mu = jnp.mean(x, axis=1, keepdims=True)
    msq = jnp.mean(x * x, axis=1, keepdims=True)
    var = msq - mu * mu
    rstd = lax.rsqrt(var + 1e-5)
    xn = (x - mu) * rstd * g_ref[...] + b_ref[...]
    xn_ref[...] = xn
    a = jnp.abs(xn)
    rows = x.shape[0]
    lo = jnp.zeros((rows, 1), jnp.float32)
    hi = jnp.full((rows, 1), float(np.sqrt(d / k)) * 1.001, jnp.float32)

    def step(_, carry):
        lo, hi = carry
        w3 = (hi - lo) * (1.0 / 3.0)
        t1 = lo + w3
        t2 = hi - w3
        cnt1 = jnp.sum(jnp.where(a >= t1, 1.0, 0.0), axis=1, keepdims=True)
        cnt2 = jnp.sum(jnp.where(a >= t2, 1.0, 0.0), axis=1, keepdims=True)
        ge1 = cnt1 >= k
        ge2 = cnt2 >= k
        lo = jnp.where(ge2, t2, jnp.where(ge1, t1, lo))
        hi = jnp.where(ge1, jnp.where(ge2, hi, t2), t1)
        return lo, hi

    lo, hi = lax.fori_loop(0, 13, step, (lo, hi))
    spikes_ref[...] = (a >= lo).astype(jnp.float32)


def _ln_topk(x, gamma, beta, block_rows=256, interpret=False):
    n, d = x.shape
    k = max(1, int(TOPK_DENSITY * d))
    g2 = gamma.reshape(1, d)
    b2 = beta.reshape(1, d)
    grid = n // block_rows
    return pl.pallas_call(
        functools.partial(_ln_topk_body, k=k),
        grid=(grid,),
        in_specs=[
            pl.BlockSpec((block_rows, d), lambda i: (i, 0)),
            pl.BlockSpec((1, d), lambda i: (0, 0)),
            pl.BlockSpec((1, d), lambda i: (0, 0)),
        ],
        out_specs=[
            pl.BlockSpec((block_rows, d), lambda i: (i, 0)),
            pl.BlockSpec((block_rows, d), lambda i: (i, 0)),
        ],
        out_shape=[
            jax.ShapeDtypeStruct((n, d), jnp.float32),
            jax.ShapeDtypeStruct((n, d), jnp.float32),
        ],
        compiler_params=pltpu.CompilerParams(
            dimension_semantics=("parallel",),
        ),
        interpret=interpret,
    )(x, g2, b2)


def kernel(token_ids, emb_table, gamma, beta):
    b, s = token_ids.shape
    v, d = emb_table.shape
    ids = token_ids.reshape(-1)
    x = _sc_gather(ids, emb_table)
    spikes, xn = _ln_topk(x, gamma, beta)
    return spikes.reshape(b, s, d), xn.reshape(b, s, d)


# speedup vs baseline: 1.0882x; 1.0882x over previous
import functools

import jax
import jax.numpy as jnp
import numpy as np
from jax import lax
from jax.experimental import pallas as pl
from jax.experimental.pallas import tpu as pltpu
from jax.experimental.pallas import tpu_sc as plsc

NC, NS = 2, 16
NW = NC * NS
GATHER_CHUNK = 32
TOPK_DENSITY = 0.11


def _sc_gather(ids, table):
    n = ids.shape[0]
    _, d = table.shape
    b_per_w = n // NW
    n_chunks = b_per_w // GATHER_CHUNK
    mesh = plsc.VectorSubcoreMesh(core_axis_name="c", subcore_axis_name="s")

    @functools.partial(
        pl.kernel,
        mesh=mesh,
        out_type=jax.ShapeDtypeStruct((n, d), jnp.float32),
        scratch_types=[
            pltpu.VMEM((GATHER_CHUNK,), jnp.int32),
            pltpu.VMEM((GATHER_CHUNK,), jnp.int32),
            pltpu.VMEM((GATHER_CHUNK, d), jnp.float32),
            pltpu.VMEM((GATHER_CHUNK, d), jnp.float32),
            pltpu.SemaphoreType.DMA,
            pltpu.SemaphoreType.DMA,
            pltpu.SemaphoreType.DMA,
            pltpu.SemaphoreType.DMA,
        ],
    )
    def gather_kernel(ids_hbm, table_hbm, out_hbm,
                      idx0, idx1, rows0, rows1, gs0, gs1, ws0, ws1):
        wid = lax.axis_index("s") * NC + lax.axis_index("c")
        base = wid * b_per_w
        idx = (idx0, idx1)
        rows = (rows0, rows1)
        gsem = (gs0, gs1)
        wsem = (ws0, ws1)
        pltpu.sync_copy(ids_hbm.at[pl.ds(base, GATHER_CHUNK)], idx0)
        gathers = {0: pltpu.async_copy(table_hbm.at[idx0], rows0, gs0)}
        writes = {}
        for i in range(n_chunks):
            cur = i % 2
            nxt = (i + 1) % 2
            if i + 1 < n_chunks:
                if i - 1 >= 0:
                    writes[i - 1].wait()
                off_n = base + (i + 1) * GATHER_CHUNK
                pltpu.sync_copy(ids_hbm.at[pl.ds(off_n, GATHER_CHUNK)],
                                idx[nxt])
                gathers[i + 1] = pltpu.async_copy(
                    table_hbm.at[idx[nxt]], rows[nxt], gsem[nxt])
            gathers[i].wait()
            off = base + i * GATHER_CHUNK
            writes[i] = pltpu.async_copy(
                rows[cur], out_hbm.at[pl.ds(off, GATHER_CHUNK)], wsem[cur])
        writes[n_chunks - 2].wait()
        writes[n_chunks - 1].wait()

    return gather_kernel(ids, table)


def _ln_topk_body(x_ref, g_ref, b_ref, spikes_ref, xn_ref, *, k):
    x = x_ref[...]
    d = x.shape[1]
    mu = jnp.mean(x, axis=1, keepdims=True)
    msq = jnp.mean(x * x, axis=1, keepdims=True)
    var = msq - mu * mu
    rstd = lax.rsqrt(var + 1e-5)
    xn = (x - mu) * rstd * g_ref[...] + b_ref[...]
    xn_ref[...] = xn
    a = jnp.abs(xn)
    rows = x.shape[0]
    lo = jnp.zeros((rows, 1), jnp.float32)
    hi = jnp.full((rows, 1), float(np.sqrt(d / k)) * 1.001, jnp.float32)

    def step(_, carry):
        lo, hi = carry
        w3 = (hi - lo) * (1.0 / 3.0)
        t1 = lo + w3
        t2 = hi - w3
        cnt1 = jnp.sum(jnp.where(a >= t1, 1.0, 0.0), axis=1, keepdims=True)
        cnt2 = jnp.sum(jnp.where(a >= t2, 1.0, 0.0), axis=1, keepdims=True)
        ge1 = cnt1 >= k
        ge2 = cnt2 >= k
        lo = jnp.where(ge2, t2, jnp.where(ge1, t1, lo))
        hi = jnp.where(ge1, jnp.where(ge2, hi, t2), t1)
        return lo, hi

    lo, hi = lax.fori_loop(0, 12, step, (lo, hi))
    spikes_ref[...] = (a >= lo).astype(jnp.float32)


def _ln_topk(x, gamma, beta, block_rows=512, interpret=False):
    n, d = x.shape
    k = max(1, int(TOPK_DENSITY * d))
    g2 = gamma.reshape(1, d)
    b2 = beta.reshape(1, d)
    grid = n // block_rows
    return pl.pallas_call(
        functools.partial(_ln_topk_body, k=k),
        grid=(grid,),
        in_specs=[
            pl.BlockSpec((block_rows, d), lambda i: (i, 0)),
            pl.BlockSpec((1, d), lambda i: (0, 0)),
            pl.BlockSpec((1, d), lambda i: (0, 0)),
        ],
        out_specs=[
            pl.BlockSpec((block_rows, d), lambda i: (i, 0)),
            pl.BlockSpec((block_rows, d), lambda i: (i, 0)),
        ],
        out_shape=[
            jax.ShapeDtypeStruct((n, d), jnp.float32),
            jax.ShapeDtypeStruct((n, d), jnp.float32),
        ],
        compiler_params=pltpu.CompilerParams(
            dimension_semantics=("parallel",),
        ),
        interpret=interpret,
    )(x, g2, b2)


def kernel(token_ids, emb_table, gamma, beta):
    b, s = token_ids.shape
    v, d = emb_table.shape
    ids = token_ids.reshape(-1)
    x = _sc_gather(ids, emb_table)
    spikes, xn = _ln_topk(x, gamma, beta)
    return spikes.reshape(b, s, d), xn.reshape(b, s, d)


# speedup vs baseline: 1.0899x; 1.0016x over previous
import functools

import jax
import jax.numpy as jnp
import numpy as np
from jax import lax
from jax.experimental import pallas as pl
from jax.experimental.pallas import tpu as pltpu
from jax.experimental.pallas import tpu_sc as plsc

NC, NS = 2, 16
NW = NC * NS
GATHER_CHUNK = 16
GATHER_RING = 4
TOPK_DENSITY = 0.11


def _sc_gather(ids, table):
    n = ids.shape[0]
    _, d = table.shape
    b_per_w = n // NW
    n_chunks = b_per_w // GATHER_CHUNK
    ring = min(GATHER_RING, n_chunks)
    mesh = plsc.VectorSubcoreMesh(core_axis_name="c", subcore_axis_name="s")

    @functools.partial(
        pl.kernel,
        mesh=mesh,
        out_type=jax.ShapeDtypeStruct((n, d), jnp.float32),
        scratch_types=(
            [pltpu.VMEM((GATHER_CHUNK,), jnp.int32) for _ in range(ring)]
            + [pltpu.VMEM((GATHER_CHUNK, d), jnp.float32) for _ in range(ring)]
            + [pltpu.SemaphoreType.DMA for _ in range(2 * ring)]
        ),
    )
    def gather_kernel(ids_hbm, table_hbm, out_hbm, *bufs):
        idx = bufs[:ring]
        rows = bufs[ring:2 * ring]
        gsem = bufs[2 * ring:3 * ring]
        wsem = bufs[3 * ring:4 * ring]
        wid = lax.axis_index("s") * NC + lax.axis_index("c")
        base = wid * b_per_w
        gathers = {}
        writes = {}
        for j in range(ring):
            pltpu.sync_copy(
                ids_hbm.at[pl.ds(base + j * GATHER_CHUNK, GATHER_CHUNK)],
                idx[j])
            gathers[j] = pltpu.async_copy(
                table_hbm.at[idx[j]], rows[j], gsem[j])
        for i in range(n_chunks):
            bslot = i % ring
            gathers[i].wait()
            writes[i] = pltpu.async_copy(
                rows[bslot],
                out_hbm.at[pl.ds(base + i * GATHER_CHUNK, GATHER_CHUNK)],
                wsem[bslot])
            j = i + ring
            if j < n_chunks:
                writes[i].wait()
                pltpu.sync_copy(
                    ids_hbm.at[pl.ds(base + j * GATHER_CHUNK, GATHER_CHUNK)],
                    idx[bslot])
                gathers[j] = pltpu.async_copy(
                    table_hbm.at[idx[bslot]], rows[bslot], gsem[bslot])
        for i in range(max(0, n_chunks - ring), n_chunks):
            writes[i].wait()

    return gather_kernel(ids, table)


def _ln_topk_body(x_ref, g_ref, b_ref, spikes_ref, xn_ref, *, k):
    x = x_ref[...]
    d = x.shape[1]
    mu = jnp.mean(x, axis=1, keepdims=True)
    msq = jnp.mean(x * x, axis=1, keepdims=True)
    var = msq - mu * mu
    rstd = lax.rsqrt(var + 1e-5)
    xn = (x - mu) * rstd * g_ref[...] + b_ref[...]
    xn_ref[...] = xn
    a = jnp.abs(xn)
    rows = x.shape[0]
    lo = jnp.zeros((rows, 1), jnp.float32)
    hi = jnp.full((rows, 1), float(np.sqrt(d / k)) * 1.001, jnp.float32)

    def step(_, carry):
        lo, hi = carry
        w3 = (hi - lo) * (1.0 / 3.0)
        t1 = lo + w3
        t2 = hi - w3
        cnt1 = jnp.sum(jnp.where(a >= t1, 1.0, 0.0), axis=1, keepdims=True)
        cnt2 = jnp.sum(jnp.where(a >= t2, 1.0, 0.0), axis=1, keepdims=True)
        ge1 = cnt1 >= k
        ge2 = cnt2 >= k
        lo = jnp.where(ge2, t2, jnp.where(ge1, t1, lo))
        hi = jnp.where(ge1, jnp.where(ge2, hi, t2), t1)
        return lo, hi

    lo, hi = lax.fori_loop(0, 12, step, (lo, hi))
    spikes_ref[...] = (a >= lo).astype(jnp.float32)


def _ln_topk(x, gamma, beta, block_rows=512, interpret=False):
    n, d = x.shape
    k = max(1, int(TOPK_DENSITY * d))
    g2 = gamma.reshape(1, d)
    b2 = beta.reshape(1, d)
    grid = n // block_rows
    return pl.pallas_call(
        functools.partial(_ln_topk_body, k=k),
        grid=(grid,),
        in_specs=[
            pl.BlockSpec((block_rows, d), lambda i: (i, 0)),
            pl.BlockSpec((1, d), lambda i: (0, 0)),
            pl.BlockSpec((1, d), lambda i: (0, 0)),
        ],
        out_specs=[
            pl.BlockSpec((block_rows, d), lambda i: (i, 0)),
            pl.BlockSpec((block_rows, d), lambda i: (i, 0)),
        ],
        out_shape=[
            jax.ShapeDtypeStruct((n, d), jnp.float32),
            jax.ShapeDtypeStruct((n, d), jnp.float32),
        ],
        compiler_params=pltpu.CompilerParams(
            dimension_semantics=("parallel",),
        ),
        interpret=interpret,
    )(x, g2, b2)


def kernel(token_ids, emb_table, gamma, beta):
    b, s = token_ids.shape
    v, d = emb_table.shape
    ids = token_ids.reshape(-1)
    x = _sc_gather(ids, emb_table)
    spikes, xn = _ln_topk(x, gamma, beta)
    return spikes.reshape(b, s, d), xn.reshape(b, s, d)
